# baseline (speedup 1.0000x reference)
import functools

import jax
import jax.numpy as jnp
from jax import lax
from jax.experimental import pallas as pl
from jax.experimental.pallas import tpu as pltpu
from jax.experimental.pallas import tpu_sc as plsc

DIM_WORD = 128
DIM_FEAT = 64
DIM_OUT = DIM_WORD + DIM_FEAT

NC = 2
NS = 16
NW = NC * NS
L = 16

BPW = 128


def _build(batch, seq):
    assert batch == NW * BPW and seq % 2 == 0

    mesh = plsc.VectorSubcoreMesh(core_axis_name="c", subcore_axis_name="s")

    @functools.partial(
        pl.kernel,
        mesh=mesh,
        out_type=jax.ShapeDtypeStruct((seq, DIM_OUT, batch), jnp.float32),
        scratch_types=[
            pltpu.VMEM((seq, BPW), jnp.int32),
            pltpu.VMEM((seq, BPW), jnp.int32),
            pltpu.VMEM((2, BPW, DIM_WORD), jnp.float32),
            pltpu.VMEM((2, BPW, DIM_WORD), jnp.float32),
            pltpu.VMEM((2, DIM_WORD, BPW), jnp.float32),
            pltpu.VMEM((2, DIM_FEAT, BPW), jnp.float32),
            pltpu.SemaphoreType.DMA,
            pltpu.SemaphoreType.DMA,
            pltpu.SemaphoreType.DMA,
            pltpu.SemaphoreType.DMA,
        ],
        compiler_params=pltpu.CompilerParams(needs_layout_passes=False),
    )
    def k(wT_hbm, fT_hbm, ww_hbm, wf_hbm, out_hbm,
          widx_v, fidx_v, gw_v, gf_v, tw_v, tf_v,
          gsem0, gsem1, wsem0, wsem1):
        wid = lax.axis_index("s") * NC + lax.axis_index("c")
        b0 = wid * BPW
        pltpu.sync_copy(wT_hbm.at[:, pl.ds(b0, BPW)], widx_v)
        pltpu.sync_copy(fT_hbm.at[:, pl.ds(b0, BPW)], fidx_v)

        gsem = (gsem0, gsem1)
        wsem = (wsem0, wsem1)
        row_ids = [lax.iota(jnp.int32, L) + L * g for g in range(BPW // L)]

        def fire_gather(s, sl):
            pltpu.async_copy(ww_hbm.at[widx_v.at[s]], gw_v.at[sl], gsem[sl])
            pltpu.async_copy(wf_hbm.at[fidx_v.at[s]], gf_v.at[sl], gsem[sl])

        def drain_gather(sl):
            pltpu.make_async_copy(ww_hbm.at[pl.ds(0, BPW)],
                                  gw_v.at[sl], gsem[sl]).wait()
            pltpu.make_async_copy(wf_hbm.at[pl.ds(0, BPW)],
                                  gf_v.at[sl], gsem[sl]).wait()

        def fire_write(s, sl):
            pltpu.async_copy(tw_v.at[sl],
                             out_hbm.at[s, pl.ds(0, DIM_WORD),
                                        pl.ds(b0, BPW)], wsem[sl])
            pltpu.async_copy(tf_v.at[sl],
                             out_hbm.at[s, pl.ds(DIM_WORD, DIM_FEAT),
                                        pl.ds(b0, BPW)], wsem[sl])

        def drain_write(sl):
            pltpu.make_async_copy(tw_v.at[sl],
                                  out_hbm.at[0, pl.ds(0, DIM_WORD),
                                             pl.ds(0, BPW)], wsem[sl]).wait()
            pltpu.make_async_copy(tf_v.at[sl],
                                  out_hbm.at[0, pl.ds(DIM_WORD, DIM_FEAT),
                                             pl.ds(0, BPW)], wsem[sl]).wait()

        def transpose(sl):
            gw, gf = gw_v.at[sl], gf_v.at[sl]
            tw, tf = tw_v.at[sl], tf_v.at[sl]

            def tr(src, dst, n_d, unroll):
                def body(b):
                    bcol = jnp.full((L,), 0, jnp.int32) + b
                    for g in range(n_d // L):
                        v = src[b, pl.ds(L * g, L)]
                        plsc.store_scatter(dst, [row_ids[g], bcol], v)
                plsc.parallel_loop(0, BPW, 1, unroll=unroll)(body)

            tr(gw, tw, DIM_WORD, 4)
            tr(gf, tf, DIM_FEAT, 4)

        def step(s, sl, next_s=None, dw=True):
            if next_s is not None:
                fire_gather(next_s, 1 - sl)
            drain_gather(sl)
            if dw:
                drain_write(sl)
            transpose(sl)
            fire_write(s, sl)

        fire_gather(0, 0)
        step(0, 0, next_s=1, dw=False)
        step(1, 1, next_s=2, dw=False)

        def body(k_, _):
            s0 = 2 * k_
            step(s0, 0, next_s=s0 + 1)
            step(s0 + 1, 1, next_s=s0 + 2)
            return _

        lax.fori_loop(1, seq // 2 - 1, body, 0)

        step(seq - 2, 0, next_s=seq - 1)
        step(seq - 1, 1)
        drain_write(0)
        drain_write(1)

    return k


def kernel(word, feat, W_word, W_feat):
    b, s = word.shape
    wT = word.T.astype(jnp.int32)
    fT = feat.T.astype(jnp.int32)
    wf_pad = jnp.pad(W_feat, ((0, 0), (0, DIM_WORD - DIM_FEAT)))
    out = _build(b, s)(wT, fT, W_word, wf_pad)
    return out.transpose(2, 0, 1)


# speedup vs baseline: 3.1589x; 3.1589x over previous
import functools

import jax
import jax.numpy as jnp
from jax import lax
from jax.experimental import pallas as pl
from jax.experimental.pallas import tpu as pltpu
from jax.experimental.pallas import tpu_sc as plsc

DIM_WORD = 128
DIM_FEAT = 64
DIM_OUT = DIM_WORD + DIM_FEAT

NC = 2
NS = 16
NW = NC * NS
L = 16

BPW = 128


def _build(batch, seq):
    assert batch == NW * BPW and seq % 2 == 0

    mesh = plsc.VectorSubcoreMesh(core_axis_name="c", subcore_axis_name="s")

    @functools.partial(
        pl.kernel,
        mesh=mesh,
        out_type=jax.ShapeDtypeStruct((seq, DIM_OUT, batch), jnp.float32),
        scratch_types=[
            pltpu.VMEM((seq, BPW), jnp.int32),
            pltpu.VMEM((seq, BPW), jnp.int32),
            pltpu.VMEM((2, BPW, DIM_WORD), jnp.float32),
            pltpu.VMEM((2, BPW, DIM_WORD), jnp.float32),
            pltpu.VMEM((2, DIM_WORD, BPW), jnp.float32),
            pltpu.VMEM((2, DIM_FEAT, BPW), jnp.float32),
            pltpu.SemaphoreType.DMA,
            pltpu.SemaphoreType.DMA,
            pltpu.SemaphoreType.DMA,
            pltpu.SemaphoreType.DMA,
        ],
        compiler_params=pltpu.CompilerParams(needs_layout_passes=False),
    )
    def k(wT_hbm, fT_hbm, ww_hbm, wf_hbm, out_hbm,
          widx_v, fidx_v, gw_v, gf_v, tw_v, tf_v,
          gsem0, gsem1, wsem0, wsem1):
        wid = lax.axis_index("s") * NC + lax.axis_index("c")
        b0 = wid * BPW
        pltpu.sync_copy(wT_hbm.at[:, pl.ds(b0, BPW)], widx_v)
        pltpu.sync_copy(fT_hbm.at[:, pl.ds(b0, BPW)], fidx_v)

        gsem = (gsem0, gsem1)
        wsem = (wsem0, wsem1)
        iota = lax.iota(jnp.int32, L)
        row_ids = [iota + L * g for g in range(BPW // L)]

        def fire_gather(s, sl):
            pltpu.async_copy(ww_hbm.at[widx_v.at[s]], gw_v.at[sl], gsem[sl])
            pltpu.async_copy(wf_hbm.at[fidx_v.at[s]], gf_v.at[sl], gsem[sl])

        def drain_gather(sl):
            pltpu.make_async_copy(ww_hbm.at[pl.ds(0, BPW)],
                                  gw_v.at[sl], gsem[sl]).wait()
            pltpu.make_async_copy(wf_hbm.at[pl.ds(0, BPW)],
                                  gf_v.at[sl], gsem[sl]).wait()

        def fire_write(s, sl):
            pltpu.async_copy(tw_v.at[sl],
                             out_hbm.at[s, pl.ds(0, DIM_WORD),
                                        pl.ds(b0, BPW)], wsem[sl])
            pltpu.async_copy(tf_v.at[sl],
                             out_hbm.at[s, pl.ds(DIM_WORD, DIM_FEAT),
                                        pl.ds(b0, BPW)], wsem[sl])

        def drain_write(sl):
            pltpu.make_async_copy(tw_v.at[sl],
                                  out_hbm.at[0, pl.ds(0, DIM_WORD),
                                             pl.ds(0, BPW)], wsem[sl]).wait()
            pltpu.make_async_copy(tf_v.at[sl],
                                  out_hbm.at[0, pl.ds(DIM_WORD, DIM_FEAT),
                                             pl.ds(0, BPW)], wsem[sl]).wait()

        def transpose(sl):
            gw, gf = gw_v.at[sl], gf_v.at[sl]
            tw, tf = tw_v.at[sl], tf_v.at[sl]

            def tr(src, dst, n_d, unroll):
                def body(it):
                    rot = (iota + it) & (L - 1)
                    c_idx = rot + (it & -L)
                    for g in range(BPW // L):
                        v = plsc.load_gather(src, [row_ids[g], c_idx])
                        plsc.store_scatter(dst, [c_idx, row_ids[g]], v)
                plsc.parallel_loop(0, n_d, 1, unroll=unroll)(body)

            tr(gw, tw, DIM_WORD, 2)
            tr(gf, tf, DIM_FEAT, 2)

        def step(s, sl, next_s=None, dw=True):
            if next_s is not None:
                fire_gather(next_s, 1 - sl)
            drain_gather(sl)
            if dw:
                drain_write(sl)
            transpose(sl)
            fire_write(s, sl)

        fire_gather(0, 0)
        step(0, 0, next_s=1, dw=False)
        step(1, 1, next_s=2, dw=False)

        def body(k_, _):
            s0 = 2 * k_
            step(s0, 0, next_s=s0 + 1)
            step(s0 + 1, 1, next_s=s0 + 2)
            return _

        lax.fori_loop(1, seq // 2 - 1, body, 0)

        step(seq - 2, 0, next_s=seq - 1)
        step(seq - 1, 1)
        drain_write(0)
        drain_write(1)

    return k


def kernel(word, feat, W_word, W_feat):
    b, s = word.shape
    wT = word.T.astype(jnp.int32)
    fT = feat.T.astype(jnp.int32)
    wf_pad = jnp.pad(W_feat, ((0, 0), (0, DIM_WORD - DIM_FEAT)))
    out = _build(b, s)(wT, fT, W_word, wf_pad)
    return out.transpose(2, 0, 1)
